# baseline (speedup 1.0000x reference)
import functools

import jax
import jax.numpy as jnp
from jax import lax
from jax.experimental import pallas as pl
from jax.experimental.pallas import tpu as pltpu
from jax.experimental.pallas import tpu_sc as plsc

_ALPHA = 0.5
_B = 1024
_C = 3129
_D = 32
_BT = 512
_NB = 16
_NW = 32
_ROWS = (_B - _BT) // _NW
_CHUNK_G = 28
_NCHUNK = 7
_CHUNK_J = _CHUNK_G * 16
_CHUNK_W = _CHUNK_J * _D
_LAST_W = (_C - (_NCHUNK - 1) * _CHUNK_J) * _D
_BIG_I = 1 << 30
_INF = float("inf")


def _hreduce(x, op):
    lane = lax.iota(jnp.int32, 16)
    for s in (8, 4, 2, 1):
        x = op(x, x.at[lane ^ s].get(mode="promise_in_bounds",
                                     unique_indices=True))
    return x[0]


def _labels_stats(lab_v):
    lane = lax.iota(jnp.int32, 16)
    nvec = 3200 // 16

    def pass1(i, acc):
        return jnp.maximum(acc, lab_v[pl.ds(i * 16, 16)])

    macc = lax.fori_loop(0, nvec, pass1, jnp.full((16,), -_INF, jnp.float32))
    m = _hreduce(macc, jnp.maximum)

    def pass2(i, carry):
        fidx, lidx = carry
        v = lab_v[pl.ds(i * 16, 16)]
        jv = i * 16 + lane
        is_m = v == m
        fidx = jnp.minimum(fidx, jnp.where(is_m, jv, _BIG_I))
        lidx = jnp.maximum(lidx, jnp.where(is_m, jv, jnp.int32(-1)))
        return fidx, lidx

    fidx, lidx = lax.fori_loop(
        0, nvec, pass2,
        (jnp.full((16,), _BIG_I, jnp.int32), jnp.full((16,), -1, jnp.int32)))
    return m, _hreduce(fidx, jnp.minimum), _hreduce(lidx, jnp.maximum)


def _chunk_fold(a_v, rots, cbase, first_idx, last_idx, corr_acc, min_acc):
    lane = lax.iota(jnp.int32, 16)
    lane32 = lane * _D

    def group(g, carry):
        corr_acc, min_acc = carry
        base = g * (16 * _D) + lane32
        acc = jnp.zeros((16,), jnp.float32)
        for t in range(_D):
            idx = base + ((t + lane) & (_D - 1))
            diff = plsc.load_gather(a_v, [idx]) - rots[t]
            acc = acc + diff * diff
        jg = cbase + g * 16 + lane
        corr_acc = corr_acc + jnp.where(jg == first_idx, acc, 0.0)
        keep = (jg != last_idx) & (jg < _C)
        min_acc = jnp.minimum(min_acc, jnp.where(keep, acc, _INF))
        return corr_acc, min_acc

    return lax.fori_loop(0, _CHUNK_G, group, (corr_acc, min_acc))


def _sc_body(ans_hbm, labels_hbm, logits_hbm, out_hbm,
             lab_v, lg_v, abuf0, abuf1, ov, sem0, sem1):
    wid = lax.axis_index("s") * 2 + lax.axis_index("c")
    abufs = (abuf0, abuf1)
    sems = (sem0, sem1)

    def a_copy(b, c):
        nw = _CHUNK_W if c < _NCHUNK - 1 else _LAST_W
        return pltpu.make_async_copy(
            ans_hbm.at[pl.ds(b * (_C * _D) + c * _CHUNK_W, nw)],
            abufs[c % 2].at[pl.ds(0, nw)],
            sems[c % 2])

    def row(r, total):
        b = _BT + wid * _ROWS + r
        a_copy(b, 0).start()
        a_copy(b, 1).start()
        pltpu.sync_copy(labels_hbm.at[b], lab_v)
        pltpu.sync_copy(logits_hbm.at[b], lg_v)
        lane = lax.iota(jnp.int32, 16)
        rots = [plsc.load_gather(lg_v, [(t + lane) & (_D - 1)])
                for t in range(_D)]
        m, first_idx, last_idx = _labels_stats(lab_v)

        corr_acc = jnp.zeros((16,), jnp.float32)
        min_acc = jnp.full((16,), _INF, jnp.float32)
        for c in range(_NCHUNK):
            a_copy(b, c).wait()
            corr_acc, min_acc = _chunk_fold(
                abufs[c % 2], rots, c * _CHUNK_J,
                first_idx, last_idx, corr_acc, min_acc)
            if c + 2 < _NCHUNK:
                a_copy(b, c + 2).start()

        d_correct = _hreduce(corr_acc, jnp.add)
        hardest = _hreduce(min_acc, jnp.minimum)
        return total + jnp.maximum(d_correct - _ALPHA * hardest, 0.0)

    total = lax.fori_loop(0, _ROWS, row, jnp.float32(0.0))
    ov[...] = jnp.full((16,), total, jnp.float32)
    pltpu.sync_copy(ov, out_hbm.at[wid])


def _tc_body(logits_ref, labels_ref, emb_ref, out_ref):
    step = pl.program_id(0)

    A = emb_ref[...]
    l = logits_ref[...]
    lab = labels_ref[...]
    C = lab.shape[2]

    G = A * (A - 2.0 * l)
    ones = jnp.ones((l.shape[0], 1, l.shape[2]), jnp.float32)
    dots = jax.lax.dot_general(
        ones, G,
        dimension_numbers=(((2,), (2,)), ((0,), (0,))),
        preferred_element_type=jnp.float32)
    lsq = jnp.sum(l * l, axis=2, keepdims=True)
    d = dots + lsq

    m = jnp.max(lab, axis=2, keepdims=True)
    iota_l = jax.lax.broadcasted_iota(jnp.int32, lab.shape, 2)
    is_max = lab == m
    first_idx = jnp.min(jnp.where(is_max, iota_l, C), axis=2, keepdims=True)
    last_idx = jnp.max(jnp.where(is_max, iota_l, -1), axis=2, keepdims=True)

    d_correct = jnp.sum(
        jnp.where(iota_l == first_idx, d, 0.0), axis=2, keepdims=True)
    hardest = jnp.min(
        jnp.where(iota_l == last_idx, jnp.float32(jnp.inf), d),
        axis=2, keepdims=True)
    loss = jnp.sum(jnp.maximum(d_correct - _ALPHA * hardest, 0.0))

    @pl.when(step == 0)
    def _init():
        out_ref[...] = jnp.zeros_like(out_ref)

    out_ref[...] = out_ref[...] + loss


def kernel(logits, labels, ans_emb, print_info):
    B, C = labels.shape
    D = logits.shape[1]

    ans_flat = ans_emb.reshape(B * C * D)
    labels_p = jnp.pad(labels, ((0, 0), (0, 3200 - C)), constant_values=-1.0)
    mesh = plsc.VectorSubcoreMesh(core_axis_name="c", subcore_axis_name="s")
    sc_run = functools.partial(
        pl.kernel, _sc_body, mesh=mesh,
        compiler_params=pltpu.CompilerParams(needs_layout_passes=False),
        out_type=jax.ShapeDtypeStruct((_NW, 16), jnp.float32),
        scratch_types=[
            pltpu.VMEM((3200,), jnp.float32),
            pltpu.VMEM((_D,), jnp.float32),
            pltpu.VMEM((_CHUNK_W,), jnp.float32),
            pltpu.VMEM((_CHUNK_W,), jnp.float32),
            pltpu.VMEM((16,), jnp.float32),
            pltpu.SemaphoreType.DMA,
            pltpu.SemaphoreType.DMA,
        ],
    )()
    sc_partials = sc_run(ans_flat, labels_p, logits)

    tc_out = pl.pallas_call(
        _tc_body,
        grid=(_BT // _NB,),
        in_specs=[
            pl.BlockSpec((_NB, 1, D), lambda i: (i, 0, 0)),
            pl.BlockSpec((_NB, 1, C), lambda i: (i, 0, 0)),
            pl.BlockSpec((_NB, C, D), lambda i: (i, 0, 0)),
        ],
        out_specs=pl.BlockSpec((1, 1), lambda i: (0, 0)),
        out_shape=jax.ShapeDtypeStruct((1, 1), jnp.float32),
    )(logits.reshape(B, 1, D), labels.reshape(B, 1, C), ans_emb)

    return tc_out[0, 0] + jnp.sum(sc_partials[:, 0])


# speedup vs baseline: 1.4956x; 1.4956x over previous
import functools

import jax
import jax.numpy as jnp
from jax import lax
from jax.experimental import pallas as pl
from jax.experimental.pallas import tpu as pltpu
from jax.experimental.pallas import tpu_sc as plsc

_ALPHA = 0.5
_B = 1024
_C = 3129
_D = 32
_NW = 32
_ROWS = _B // _NW
_GROUPS = 196
_CHUNK_G = 28
_NCHUNK = 7
_CHUNK_J = _CHUNK_G * 16
_CHUNK_W = _CHUNK_J * _D
_LAST_W = (_C - (_NCHUNK - 1) * _CHUNK_J) * _D
_BIG_I = 1 << 30
_INF = float("inf")


def _hreduce(x, op):
    lane = lax.iota(jnp.int32, 16)
    for s in (8, 4, 2, 1):
        x = op(x, x.at[lane ^ s].get(mode="promise_in_bounds",
                                     unique_indices=True))
    return x[0]


def _labels_stats(lab_v):
    lane = lax.iota(jnp.int32, 16)
    nvec = 3200 // 16

    def pass1(i, acc):
        return jnp.maximum(acc, lab_v[pl.ds(i * 16, 16)])

    macc = lax.fori_loop(0, nvec, pass1, jnp.full((16,), -_INF, jnp.float32))
    m = _hreduce(macc, jnp.maximum)

    def pass2(i, carry):
        fidx, lidx = carry
        v = lab_v[pl.ds(i * 16, 16)]
        jv = i * 16 + lane
        is_m = v == m
        fidx = jnp.minimum(fidx, jnp.where(is_m, jv, _BIG_I))
        lidx = jnp.maximum(lidx, jnp.where(is_m, jv, jnp.int32(-1)))
        return fidx, lidx

    fidx, lidx = lax.fori_loop(
        0, nvec, pass2,
        (jnp.full((16,), _BIG_I, jnp.int32), jnp.full((16,), -1, jnp.int32)))
    return m, _hreduce(fidx, jnp.minimum), _hreduce(lidx, jnp.maximum)


def _chunk_fold(a_v, rots, cbase, first_idx, last_idx, corr_acc, min_acc):
    lane = lax.iota(jnp.int32, 16)
    lane32 = lane * _D

    def group(g, carry):
        corr_acc, min_acc = carry
        base = g * (16 * _D) + lane32
        acc = jnp.zeros((16,), jnp.float32)
        for t in range(_D):
            idx = base + ((t + lane) & (_D - 1))
            diff = plsc.load_gather(a_v, [idx]) - rots[t]
            acc = acc + diff * diff
        jg = cbase + g * 16 + lane
        corr_acc = corr_acc + jnp.where(jg == first_idx, acc, 0.0)
        keep = (jg != last_idx) & (jg < _C)
        min_acc = jnp.minimum(min_acc, jnp.where(keep, acc, _INF))
        return corr_acc, min_acc

    return lax.fori_loop(0, _CHUNK_G, group, (corr_acc, min_acc))


def _sc_body(ans_hbm, labels_hbm, logits_hbm, out_hbm,
             lab_v, lg_v, abuf0, abuf1, ov, sem0, sem1):
    wid = lax.axis_index("s") * 2 + lax.axis_index("c")
    abufs = (abuf0, abuf1)
    sems = (sem0, sem1)

    def a_copy(b, c):
        nw = _CHUNK_W if c < _NCHUNK - 1 else _LAST_W
        return pltpu.make_async_copy(
            ans_hbm.at[pl.ds(b * (_C * _D) + c * _CHUNK_W, nw)],
            abufs[c % 2].at[pl.ds(0, nw)],
            sems[c % 2])

    def row(r, total):
        b = wid * _ROWS + r
        a_copy(b, 0).start()
        a_copy(b, 1).start()
        pltpu.sync_copy(labels_hbm.at[b], lab_v)
        pltpu.sync_copy(logits_hbm.at[b], lg_v)
        lane = lax.iota(jnp.int32, 16)
        rots = [plsc.load_gather(lg_v, [(t + lane) & (_D - 1)])
                for t in range(_D)]
        m, first_idx, last_idx = _labels_stats(lab_v)

        corr_acc = jnp.zeros((16,), jnp.float32)
        min_acc = jnp.full((16,), _INF, jnp.float32)
        for c in range(_NCHUNK):
            a_copy(b, c).wait()
            corr_acc, min_acc = _chunk_fold(
                abufs[c % 2], rots, c * _CHUNK_J,
                first_idx, last_idx, corr_acc, min_acc)
            if c + 2 < _NCHUNK:
                a_copy(b, c + 2).start()

        d_correct = _hreduce(corr_acc, jnp.add)
        hardest = _hreduce(min_acc, jnp.minimum)
        return total + jnp.maximum(d_correct - _ALPHA * hardest, 0.0)

    total = lax.fori_loop(0, _ROWS, row, jnp.float32(0.0))
    ov[...] = jnp.full((16,), total, jnp.float32)
    pltpu.sync_copy(ov, out_hbm.at[wid])


def kernel(logits, labels, ans_emb, print_info):
    B, C = labels.shape
    D = logits.shape[1]
    ans_flat = ans_emb.reshape(B * C * D)
    labels_p = jnp.pad(labels, ((0, 0), (0, 3200 - C)), constant_values=-1.0)
    mesh = plsc.VectorSubcoreMesh(core_axis_name="c", subcore_axis_name="s")
    run = functools.partial(
        pl.kernel, _sc_body, mesh=mesh,
        compiler_params=pltpu.CompilerParams(needs_layout_passes=False),
        out_type=jax.ShapeDtypeStruct((_NW, 16), jnp.float32),
        scratch_types=[
            pltpu.VMEM((3200,), jnp.float32),
            pltpu.VMEM((_D,), jnp.float32),
            pltpu.VMEM((_CHUNK_W,), jnp.float32),
            pltpu.VMEM((_CHUNK_W,), jnp.float32),
            pltpu.VMEM((16,), jnp.float32),
            pltpu.SemaphoreType.DMA,
            pltpu.SemaphoreType.DMA,
        ],
    )()
    partials = run(ans_flat, labels_p, logits)
    return jnp.sum(partials[:, 0])
